# baseline (speedup 1.0000x reference)
import functools

import jax
import jax.numpy as jnp
from jax import lax
from jax.experimental import pallas as pl
from jax.experimental.pallas import tpu as pltpu
from jax.experimental.pallas import tpu_sc as plsc

N = 10000
E = 320000
G = 256
D = 128
NC = 2
NS = 16
NW = NC * NS
EPW = E // NW
CHUNK = 125
NCH = EPW // CHUNK
W = 8
NWIN = NCH // W
N_PAD = 10112
RPW = N_PAD // NS
CNT_PAD = 10240
CPW = CNT_PAD // NS
GPW = G // NW
RB = 64
RB1 = 1000

_mesh = plsc.VectorSubcoreMesh(core_axis_name="c", subcore_axis_name="s")


def _agg_body(with_cnt, *refs):
    if with_cnt:
        (x_hbm, src_hbm, dst_hbm, zrow_hbm, zcnt_hbm, ones_hbm,
         s_out, cnt_out, s_sh, cnt_sh, stg_src, stg_dst, g_a, g_b, ones_v,
         sem_a, sem_b, sem_i, sem_sa, sem_sb, sem_c) = refs
    else:
        (x_hbm, src_hbm, dst_hbm, zrow_hbm,
         s_out, s_sh, stg_src, stg_dst, g_a, g_b,
         sem_a, sem_b, sem_i, sem_sa, sem_sb) = refs
    cid = lax.axis_index("c")
    sid = lax.axis_index("s")
    wid = cid * NS + sid

    pltpu.sync_copy(zrow_hbm.at[pl.ds(sid * RPW, RPW)],
                    s_sh.at[pl.ds(sid * RPW, RPW)])
    if with_cnt:
        pltpu.sync_copy(zcnt_hbm, cnt_sh.at[pl.ds(sid * CPW, CPW)])
        pltpu.sync_copy(ones_hbm, ones_v)
    plsc.subcore_barrier()

    def gather(p, c, buf, sem):
        pltpu.async_copy(x_hbm.at[stg_src.at[p, c]], buf, sem)

    def gwait(p, c, buf, sem):
        pltpu.make_async_copy(x_hbm.at[stg_src.at[p, c]], buf, sem).wait()

    def scatter(p, c, buf, sem):
        pltpu.async_copy(buf, s_sh.at[stg_dst.at[p, c]], sem, add=True)
        if with_cnt:
            pltpu.async_copy(ones_v, cnt_sh.at[stg_dst.at[p, c]], sem_c,
                             add=True)

    def swait(p, c, buf, sem):
        pltpu.make_async_copy(buf, s_sh.at[stg_dst.at[p, c]], sem).wait()

    pltpu.sync_copy(src_hbm.at[pl.ds(wid * NCH, W)], stg_src.at[0])
    pltpu.sync_copy(dst_hbm.at[pl.ds(wid * NCH, W)], stg_dst.at[0])
    gather(0, 0, g_a, sem_a)
    gather(0, 1, g_b, sem_b)

    def window(win, _):
        p = win % 2
        nxt = jnp.minimum(win + 1, NWIN - 1)
        nbase = wid * NCH + nxt * W
        pltpu.async_copy(src_hbm.at[pl.ds(nbase, W)], stg_src.at[1 - p], sem_i)
        pltpu.async_copy(dst_hbm.at[pl.ds(nbase, W)], stg_dst.at[1 - p], sem_i)
        for t in range(W // 2):
            c0 = 2 * t
            gwait(p, c0, g_a, sem_a)
            scatter(p, c0, g_a, sem_sa)
            gwait(p, c0 + 1, g_b, sem_b)
            scatter(p, c0 + 1, g_b, sem_sb)
            swait(p, c0, g_a, sem_sa)
            if t < W // 2 - 1:
                gather(p, c0 + 2, g_a, sem_a)
                swait(p, c0 + 1, g_b, sem_sb)
                gather(p, c0 + 3, g_b, sem_b)
            else:
                pltpu.make_async_copy(src_hbm.at[pl.ds(nbase, W)],
                                      stg_src.at[1 - p], sem_i).wait()
                pltpu.make_async_copy(dst_hbm.at[pl.ds(nbase, W)],
                                      stg_dst.at[1 - p], sem_i).wait()
                gather(1 - p, 0, g_a, sem_a)
                swait(p, c0 + 1, g_b, sem_sb)
                gather(1 - p, 1, g_b, sem_b)
        return 0

    lax.fori_loop(0, NWIN, window, 0)
    gwait(0, 0, g_a, sem_a)
    gwait(0, 1, g_b, sem_b)
    if with_cnt:
        def drain(c, _):
            pltpu.make_async_copy(ones_v, cnt_sh.at[stg_dst.at[0, 0]],
                                  sem_c).wait()
            return 0

        lax.fori_loop(0, NCH, drain, 0)
    plsc.subcore_barrier()

    r = sid * RPW
    pltpu.sync_copy(s_sh.at[pl.ds(r, RPW)], s_out.at[pl.ds(cid * N_PAD + r, RPW)])
    if with_cnt:
        pltpu.sync_copy(cnt_sh.at[pl.ds(sid * CPW, CPW)],
                        cnt_out.at[pl.ds(cid * CNT_PAD + sid * CPW, CPW)])


def _make_agg(with_cnt):
    out_type = [jax.ShapeDtypeStruct((NC * N_PAD, D), jnp.float32)]
    scratch = [
        pltpu.VMEM_SHARED((N_PAD, D), jnp.float32),
        pltpu.VMEM((2, W, CHUNK), jnp.int32),
        pltpu.VMEM((2, W, CHUNK), jnp.int32),
        pltpu.VMEM((CHUNK, D), jnp.float32),
        pltpu.VMEM((CHUNK, D), jnp.float32),
        pltpu.SemaphoreType.DMA,
        pltpu.SemaphoreType.DMA,
        pltpu.SemaphoreType.DMA,
        pltpu.SemaphoreType.DMA,
        pltpu.SemaphoreType.DMA,
    ]
    if with_cnt:
        out_type.append(jax.ShapeDtypeStruct((NC * CNT_PAD,), jnp.float32))
        scratch = ([scratch[0], pltpu.VMEM_SHARED((CNT_PAD,), jnp.float32)]
                   + scratch[1:-5]
                   + [pltpu.VMEM((CHUNK,), jnp.float32)]
                   + [pltpu.SemaphoreType.DMA] * 6)
    return pl.kernel(
        functools.partial(_agg_body, with_cnt),
        out_type=out_type,
        mesh=_mesh,
        scratch_types=scratch,
    )


def _sload(ref, idx):
    return ref[pl.ds(idx, 16)][0]


NCOARSE = (N + 15) // 16


def _pool_body(h_hbm, batch_hbm, coarse_hbm, ninf_hbm, out_hbm,
               coarse_v, bwin, rbuf, acc, sem):
    cid = lax.axis_index("c")
    sid = lax.axis_index("s")
    wid = cid * NS + sid
    g0 = wid * GPW
    pltpu.sync_copy(coarse_hbm, coarse_v.at[pl.ds(0, NCOARSE)])
    pltpu.sync_copy(ninf_hbm, acc)

    def lower_bound(tgt):
        pos = jnp.int32(0)
        w = 1024
        while w:
            cand = pos + w
            v = _sload(coarse_v, jnp.minimum(cand, NCOARSE) - 1)
            pos = jnp.where((cand <= NCOARSE) & (v < tgt), cand, pos)
            w //= 2
        return pos

    lo = jnp.maximum(lower_bound(g0) - 1, 0) * 16
    hi = jnp.minimum(lower_bound(g0 + GPW) * 16, N)

    nwin = (hi - lo + RB - 1) // RB

    def step(k, _):
        rc = pl.multiple_of(jnp.minimum(lo + k * RB, N - RB), 8)
        pltpu.sync_copy(h_hbm.at[pl.ds(rc, RB)], rbuf)
        pltpu.sync_copy(batch_hbm.at[pl.ds(rc, RB)], bwin.at[pl.ds(0, RB)])

        def row(i, _):
            g = _sload(bwin, i)
            gl = g - g0
            valid = (gl >= 0) & (gl < GPW)
            glc = jnp.clip(gl, 0, GPW - 1)
            for j in range(D // 16):
                cur = acc[glc, pl.ds(j * 16, 16)]
                new = jnp.maximum(cur, rbuf[i, pl.ds(j * 16, 16)])
                acc[glc, pl.ds(j * 16, 16)] = jnp.where(valid, new, cur)
            return 0

        lax.fori_loop(0, RB, row, 0)
        return 0

    lax.fori_loop(0, nwin, step, 0)
    pltpu.sync_copy(acc, out_hbm.at[pl.ds(g0, GPW)])


_pool = pl.kernel(
    _pool_body,
    out_type=jax.ShapeDtypeStruct((G, D), jnp.float32),
    mesh=_mesh,
    scratch_types=[
        pltpu.VMEM((NCOARSE + 31, ), jnp.int32),
        pltpu.VMEM((RB + 16,), jnp.int32),
        pltpu.VMEM((RB, D), jnp.float32),
        pltpu.VMEM((GPW, D), jnp.float32),
        pltpu.SemaphoreType.DMA,
    ],
)

_CONTRACT_T = (((1,), (1,)), ((), ()))


def _dense_block(apply_ln, s_ref, cnt_ref, x_ref, wl_ref, wr_ref, b_ref,
                 g_ref, bb_ref, o_ref):
    s = s_ref[0] + s_ref[1]
    cnt = cnt_ref[0] + cnt_ref[1]
    mean = s / jnp.maximum(cnt, 1.0)
    h = (lax.dot_general(mean, wl_ref[...], _CONTRACT_T,
                         preferred_element_type=jnp.float32)
         + lax.dot_general(x_ref[...], wr_ref[...], _CONTRACT_T,
                           preferred_element_type=jnp.float32)
         + b_ref[...])
    if apply_ln:
        mu = jnp.mean(h, axis=-1, keepdims=True)
        var = jnp.mean((h - mu) ** 2, axis=-1, keepdims=True)
        h = (h - mu) * lax.rsqrt(var + 1e-5) * g_ref[...] + bb_ref[...]
        h = jnp.maximum(h, 0.0)
    o_ref[...] = h


def _make_dense(apply_ln):
    return pl.pallas_call(
        functools.partial(_dense_block, apply_ln),
        grid=(N // RB1,),
        in_specs=[
            pl.BlockSpec((NC, RB1, D), lambda i: (0, i, 0)),
            pl.BlockSpec((NC, RB1, 1), lambda i: (0, i, 0)),
            pl.BlockSpec((RB1, D), lambda i: (i, 0)),
            pl.BlockSpec((D, D), lambda i: (0, 0)),
            pl.BlockSpec((D, D), lambda i: (0, 0)),
            pl.BlockSpec((1, D), lambda i: (0, 0)),
            pl.BlockSpec((1, D), lambda i: (0, 0)),
            pl.BlockSpec((1, D), lambda i: (0, 0)),
        ],
        out_specs=pl.BlockSpec((RB1, D), lambda i: (i, 0)),
        out_shape=jax.ShapeDtypeStruct((N, D), jnp.float32),
    )


def _head_block(p_ref, w_ref, b_ref, g_ref, bb_ref, o_ref):
    y = lax.dot_general(p_ref[...], w_ref[...], _CONTRACT_T,
                        preferred_element_type=jnp.float32) + b_ref[...]
    y = jnp.maximum(y, 0.0)
    mu = jnp.mean(y, axis=-1, keepdims=True)
    var = jnp.mean((y - mu) ** 2, axis=-1, keepdims=True)
    o_ref[...] = (y - mu) * lax.rsqrt(var + 1e-5) * g_ref[...] + bb_ref[...]


_head = pl.pallas_call(
    _head_block,
    out_shape=jax.ShapeDtypeStruct((G, D), jnp.float32),
)

_agg1 = _make_agg(True)
_agg2 = _make_agg(False)
_dense1 = _make_dense(True)
_dense2 = _make_dense(False)


def kernel(atom_features, edge_index, batch, W1l, b1, W1r, gn1, bn1,
           W2l, b2, W2r, Wout, bout, g_ln, b_ln):
    src2 = edge_index[0].reshape(E // CHUNK, CHUNK)
    dst2 = edge_index[1].reshape(E // CHUNK, CHUNK)
    zrow = jnp.zeros((N_PAD, D), jnp.float32)
    zcnt = jnp.zeros((CPW,), jnp.float32)
    ones = jnp.ones((CHUNK,), jnp.float32)
    ninf = jnp.full((GPW, D), float("-inf"), jnp.float32)

    s1, cnt = _agg1(atom_features, src2, dst2, zrow, zcnt, ones)
    s1 = s1.reshape(NC, N_PAD, D)
    cnt3 = cnt.reshape(NC, CNT_PAD, 1)
    b1r = b1.reshape(1, D)
    h1 = _dense1(s1, cnt3, atom_features, W1l, W1r, b1r,
                 gn1.reshape(1, D), bn1.reshape(1, D))
    (s2,) = _agg2(h1, src2, dst2, zrow)
    s2 = s2.reshape(NC, N_PAD, D)
    h2 = _dense2(s2, cnt3, h1, W2l, W2r, b2.reshape(1, D),
                 gn1.reshape(1, D), bn1.reshape(1, D))
    coarse = batch[::16]
    pooled = _pool(h2, batch, coarse, ninf)
    return _head(pooled, Wout, bout.reshape(1, D),
                 g_ln.reshape(1, D), b_ln.reshape(1, D))


# speedup vs baseline: 1.0491x; 1.0491x over previous
import functools

import jax
import jax.numpy as jnp
from jax import lax
from jax.experimental import pallas as pl
from jax.experimental.pallas import tpu as pltpu
from jax.experimental.pallas import tpu_sc as plsc

N = 10000
E = 320000
G = 256
D = 128
NC = 2
NS = 16
NW = NC * NS
EPW = E // NW
CHUNK = 125
NCH = EPW // CHUNK
W = 8
NWIN = NCH // W
N_PAD = 10112
RPW = N_PAD // NS
CNT_PAD = 10240
CPW = CNT_PAD // NS
GPW = G // NW
RB = 64
RB1 = 1000

_mesh = plsc.VectorSubcoreMesh(core_axis_name="c", subcore_axis_name="s")


def _agg_body(with_cnt, *refs):
    if with_cnt:
        (x_hbm, src_hbm, dst_hbm, zrow_hbm, zcnt_hbm, ones_hbm,
         s_out, cnt_out, s_sh, cnt_sh, stg_src, stg_dst, g_a, g_b, ones_v,
         sem_a, sem_b, sem_i, sem_sa, sem_sb, sem_c) = refs
    else:
        (x_hbm, src_hbm, dst_hbm, zrow_hbm,
         s_out, s_sh, stg_src, stg_dst, g_a, g_b,
         sem_a, sem_b, sem_i, sem_sa, sem_sb) = refs
    cid = lax.axis_index("c")
    sid = lax.axis_index("s")
    wid = cid * NS + sid

    pltpu.sync_copy(zrow_hbm.at[pl.ds(sid * RPW, RPW)],
                    s_sh.at[pl.ds(sid * RPW, RPW)])
    if with_cnt:
        pltpu.sync_copy(zcnt_hbm, cnt_sh.at[pl.ds(sid * CPW, CPW)])
        pltpu.sync_copy(ones_hbm, ones_v)
    plsc.subcore_barrier()

    def gather(p, c, buf, sem):
        pltpu.async_copy(x_hbm.at[stg_src.at[p, c]], buf, sem)

    def gwait(p, c, buf, sem):
        pltpu.make_async_copy(x_hbm.at[stg_src.at[p, c]], buf, sem).wait()

    def scatter(p, c, buf):
        pltpu.sync_copy(buf, s_sh.at[stg_dst.at[p, c]], add=True)
        if with_cnt:
            pltpu.async_copy(ones_v, cnt_sh.at[stg_dst.at[p, c]], sem_c,
                             add=True)

    pltpu.sync_copy(src_hbm.at[pl.ds(wid * NCH, W)], stg_src.at[0])
    pltpu.sync_copy(dst_hbm.at[pl.ds(wid * NCH, W)], stg_dst.at[0])
    gather(0, 0, g_a, sem_a)

    def window(win, _):
        p = win % 2
        nxt = jnp.minimum(win + 1, NWIN - 1)
        nbase = wid * NCH + nxt * W
        pltpu.async_copy(src_hbm.at[pl.ds(nbase, W)], stg_src.at[1 - p], sem_i)
        pltpu.async_copy(dst_hbm.at[pl.ds(nbase, W)], stg_dst.at[1 - p], sem_i)
        for t in range(W // 2 - 1):
            c0 = 2 * t
            gwait(p, c0, g_a, sem_a)
            gather(p, c0 + 1, g_b, sem_b)
            scatter(p, c0, g_a)
            gwait(p, c0 + 1, g_b, sem_b)
            gather(p, c0 + 2, g_a, sem_a)
            scatter(p, c0 + 1, g_b)
        gwait(p, W - 2, g_a, sem_a)
        gather(p, W - 1, g_b, sem_b)
        scatter(p, W - 2, g_a)
        gwait(p, W - 1, g_b, sem_b)
        pltpu.make_async_copy(src_hbm.at[pl.ds(nbase, W)], stg_src.at[1 - p],
                              sem_i).wait()
        pltpu.make_async_copy(dst_hbm.at[pl.ds(nbase, W)], stg_dst.at[1 - p],
                              sem_i).wait()
        gather(1 - p, 0, g_a, sem_a)
        scatter(p, W - 1, g_b)
        return 0

    lax.fori_loop(0, NWIN, window, 0)
    gwait(0, 0, g_a, sem_a)
    if with_cnt:
        def drain(c, _):
            pltpu.make_async_copy(ones_v, cnt_sh.at[stg_dst.at[0, 0]],
                                  sem_c).wait()
            return 0

        lax.fori_loop(0, NCH, drain, 0)
    plsc.subcore_barrier()

    r = sid * RPW
    pltpu.sync_copy(s_sh.at[pl.ds(r, RPW)], s_out.at[pl.ds(cid * N_PAD + r, RPW)])
    if with_cnt:
        pltpu.sync_copy(cnt_sh.at[pl.ds(sid * CPW, CPW)],
                        cnt_out.at[pl.ds(cid * CNT_PAD + sid * CPW, CPW)])


def _make_agg(with_cnt):
    out_type = [jax.ShapeDtypeStruct((NC * N_PAD, D), jnp.float32)]
    scratch = [
        pltpu.VMEM_SHARED((N_PAD, D), jnp.float32),
        pltpu.VMEM((2, W, CHUNK), jnp.int32),
        pltpu.VMEM((2, W, CHUNK), jnp.int32),
        pltpu.VMEM((CHUNK, D), jnp.float32),
        pltpu.VMEM((CHUNK, D), jnp.float32),
        pltpu.SemaphoreType.DMA,
        pltpu.SemaphoreType.DMA,
        pltpu.SemaphoreType.DMA,
        pltpu.SemaphoreType.DMA,
        pltpu.SemaphoreType.DMA,
    ]
    if with_cnt:
        out_type.append(jax.ShapeDtypeStruct((NC * CNT_PAD,), jnp.float32))
        scratch = ([scratch[0], pltpu.VMEM_SHARED((CNT_PAD,), jnp.float32)]
                   + scratch[1:-5]
                   + [pltpu.VMEM((CHUNK,), jnp.float32)]
                   + [pltpu.SemaphoreType.DMA] * 6)
    return pl.kernel(
        functools.partial(_agg_body, with_cnt),
        out_type=out_type,
        mesh=_mesh,
        scratch_types=scratch,
    )


def _sload(ref, idx):
    return ref[pl.ds(idx, 16)][0]


NCOARSE = (N + 15) // 16


def _pool_body(h_hbm, batch_hbm, coarse_hbm, ninf_hbm, out_hbm,
               coarse_v, bwin, rbuf, acc, sem):
    cid = lax.axis_index("c")
    sid = lax.axis_index("s")
    wid = cid * NS + sid
    g0 = wid * GPW
    pltpu.sync_copy(coarse_hbm, coarse_v.at[pl.ds(0, NCOARSE)])
    pltpu.sync_copy(ninf_hbm, acc)

    def lower_bound(tgt):
        pos = jnp.int32(0)
        w = 1024
        while w:
            cand = pos + w
            v = _sload(coarse_v, jnp.minimum(cand, NCOARSE) - 1)
            pos = jnp.where((cand <= NCOARSE) & (v < tgt), cand, pos)
            w //= 2
        return pos

    lo = jnp.maximum(lower_bound(g0) - 1, 0) * 16
    hi = jnp.minimum(lower_bound(g0 + GPW) * 16, N)

    nwin = (hi - lo + RB - 1) // RB

    def step(k, _):
        rc = pl.multiple_of(jnp.minimum(lo + k * RB, N - RB), 8)
        pltpu.sync_copy(h_hbm.at[pl.ds(rc, RB)], rbuf)
        pltpu.sync_copy(batch_hbm.at[pl.ds(rc, RB)], bwin.at[pl.ds(0, RB)])

        def row(i, _):
            g = _sload(bwin, i)
            gl = g - g0
            valid = (gl >= 0) & (gl < GPW)
            glc = jnp.clip(gl, 0, GPW - 1)
            for j in range(D // 16):
                cur = acc[glc, pl.ds(j * 16, 16)]
                new = jnp.maximum(cur, rbuf[i, pl.ds(j * 16, 16)])
                acc[glc, pl.ds(j * 16, 16)] = jnp.where(valid, new, cur)
            return 0

        lax.fori_loop(0, RB, row, 0)
        return 0

    lax.fori_loop(0, nwin, step, 0)
    pltpu.sync_copy(acc, out_hbm.at[pl.ds(g0, GPW)])


_pool = pl.kernel(
    _pool_body,
    out_type=jax.ShapeDtypeStruct((G, D), jnp.float32),
    mesh=_mesh,
    scratch_types=[
        pltpu.VMEM((NCOARSE + 31, ), jnp.int32),
        pltpu.VMEM((RB + 16,), jnp.int32),
        pltpu.VMEM((RB, D), jnp.float32),
        pltpu.VMEM((GPW, D), jnp.float32),
        pltpu.SemaphoreType.DMA,
    ],
)

_CONTRACT_T = (((1,), (1,)), ((), ()))


def _dense_block(apply_ln, s_ref, cnt_ref, x_ref, wl_ref, wr_ref, b_ref,
                 g_ref, bb_ref, o_ref):
    s = s_ref[0] + s_ref[1]
    cnt = cnt_ref[0] + cnt_ref[1]
    mean = s / jnp.maximum(cnt, 1.0)
    h = (lax.dot_general(mean, wl_ref[...], _CONTRACT_T,
                         preferred_element_type=jnp.float32)
         + lax.dot_general(x_ref[...], wr_ref[...], _CONTRACT_T,
                           preferred_element_type=jnp.float32)
         + b_ref[...])
    if apply_ln:
        mu = jnp.mean(h, axis=-1, keepdims=True)
        var = jnp.mean((h - mu) ** 2, axis=-1, keepdims=True)
        h = (h - mu) * lax.rsqrt(var + 1e-5) * g_ref[...] + bb_ref[...]
        h = jnp.maximum(h, 0.0)
    o_ref[...] = h


def _make_dense(apply_ln):
    return pl.pallas_call(
        functools.partial(_dense_block, apply_ln),
        grid=(N // RB1,),
        in_specs=[
            pl.BlockSpec((NC, RB1, D), lambda i: (0, i, 0)),
            pl.BlockSpec((NC, RB1, 1), lambda i: (0, i, 0)),
            pl.BlockSpec((RB1, D), lambda i: (i, 0)),
            pl.BlockSpec((D, D), lambda i: (0, 0)),
            pl.BlockSpec((D, D), lambda i: (0, 0)),
            pl.BlockSpec((1, D), lambda i: (0, 0)),
            pl.BlockSpec((1, D), lambda i: (0, 0)),
            pl.BlockSpec((1, D), lambda i: (0, 0)),
        ],
        out_specs=pl.BlockSpec((RB1, D), lambda i: (i, 0)),
        out_shape=jax.ShapeDtypeStruct((N, D), jnp.float32),
    )


def _head_block(p_ref, w_ref, b_ref, g_ref, bb_ref, o_ref):
    y = lax.dot_general(p_ref[...], w_ref[...], _CONTRACT_T,
                        preferred_element_type=jnp.float32) + b_ref[...]
    y = jnp.maximum(y, 0.0)
    mu = jnp.mean(y, axis=-1, keepdims=True)
    var = jnp.mean((y - mu) ** 2, axis=-1, keepdims=True)
    o_ref[...] = (y - mu) * lax.rsqrt(var + 1e-5) * g_ref[...] + bb_ref[...]


_head = pl.pallas_call(
    _head_block,
    out_shape=jax.ShapeDtypeStruct((G, D), jnp.float32),
)

_agg1 = _make_agg(True)
_agg2 = _make_agg(False)
_dense1 = _make_dense(True)
_dense2 = _make_dense(False)


def kernel(atom_features, edge_index, batch, W1l, b1, W1r, gn1, bn1,
           W2l, b2, W2r, Wout, bout, g_ln, b_ln):
    src2 = edge_index[0].reshape(E // CHUNK, CHUNK)
    dst2 = edge_index[1].reshape(E // CHUNK, CHUNK)
    zrow = jnp.zeros((N_PAD, D), jnp.float32)
    zcnt = jnp.zeros((CPW,), jnp.float32)
    ones = jnp.ones((CHUNK,), jnp.float32)
    ninf = jnp.full((GPW, D), float("-inf"), jnp.float32)

    s1, cnt = _agg1(atom_features, src2, dst2, zrow, zcnt, ones)
    s1 = s1.reshape(NC, N_PAD, D)
    cnt3 = cnt.reshape(NC, CNT_PAD, 1)
    b1r = b1.reshape(1, D)
    h1 = _dense1(s1, cnt3, atom_features, W1l, W1r, b1r,
                 gn1.reshape(1, D), bn1.reshape(1, D))
    (s2,) = _agg2(h1, src2, dst2, zrow)
    s2 = s2.reshape(NC, N_PAD, D)
    h2 = _dense2(s2, cnt3, h1, W2l, W2r, b2.reshape(1, D),
                 gn1.reshape(1, D), bn1.reshape(1, D))
    coarse = batch[::16]
    pooled = _pool(h2, batch, coarse, ninf)
    return _head(pooled, Wout, bout.reshape(1, D),
                 g_ln.reshape(1, D), b_ln.reshape(1, D))


# speedup vs baseline: 1.0686x; 1.0186x over previous
import functools

import jax
import jax.numpy as jnp
from jax import lax
from jax.experimental import pallas as pl
from jax.experimental.pallas import tpu as pltpu
from jax.experimental.pallas import tpu_sc as plsc

N = 10000
E = 320000
G = 256
D = 128
NC = 2
NS = 16
NW = NC * NS
EPW = E // NW
CHUNK = 125
NCH = EPW // CHUNK
W = 8
NWIN = NCH // W
N_PAD = 10112
RPW = N_PAD // NS
CNT_PAD = 10240
CPW = CNT_PAD // NS
GPW = G // NW
RB = 128
RB1 = 2000

_mesh = plsc.VectorSubcoreMesh(core_axis_name="c", subcore_axis_name="s")


def _agg_body(with_cnt, *refs):
    if with_cnt:
        (x_hbm, src_hbm, dst_hbm, zrow_hbm, zcnt_hbm, ones_hbm,
         s_out, cnt_out, s_sh, cnt_sh, stg_src, stg_dst, g_a, g_b, ones_v,
         sem_a, sem_b, sem_i, sem_sa, sem_sb, sem_c) = refs
    else:
        (x_hbm, src_hbm, dst_hbm, zrow_hbm,
         s_out, s_sh, stg_src, stg_dst, g_a, g_b,
         sem_a, sem_b, sem_i, sem_sa, sem_sb) = refs
    cid = lax.axis_index("c")
    sid = lax.axis_index("s")
    wid = cid * NS + sid

    pltpu.sync_copy(zrow_hbm.at[pl.ds(sid * RPW, RPW)],
                    s_sh.at[pl.ds(sid * RPW, RPW)])
    if with_cnt:
        pltpu.sync_copy(zcnt_hbm, cnt_sh.at[pl.ds(sid * CPW, CPW)])
        pltpu.sync_copy(ones_hbm, ones_v)
    plsc.subcore_barrier()

    def gather(p, c, buf, sem):
        pltpu.async_copy(x_hbm.at[stg_src.at[p, c]], buf, sem)

    def gwait(p, c, buf, sem):
        pltpu.make_async_copy(x_hbm.at[stg_src.at[p, c]], buf, sem).wait()

    def scatter(p, c, buf):
        pltpu.sync_copy(buf, s_sh.at[stg_dst.at[p, c]], add=True)
        if with_cnt:
            pltpu.async_copy(ones_v, cnt_sh.at[stg_dst.at[p, c]], sem_c,
                             add=True)

    pltpu.sync_copy(src_hbm.at[pl.ds(wid * NCH, W)], stg_src.at[0])
    pltpu.sync_copy(dst_hbm.at[pl.ds(wid * NCH, W)], stg_dst.at[0])
    gather(0, 0, g_a, sem_a)

    def window(win, _):
        p = win % 2
        nxt = jnp.minimum(win + 1, NWIN - 1)
        nbase = wid * NCH + nxt * W
        pltpu.async_copy(src_hbm.at[pl.ds(nbase, W)], stg_src.at[1 - p], sem_i)
        pltpu.async_copy(dst_hbm.at[pl.ds(nbase, W)], stg_dst.at[1 - p], sem_i)
        for t in range(W // 2 - 1):
            c0 = 2 * t
            gwait(p, c0, g_a, sem_a)
            gather(p, c0 + 1, g_b, sem_b)
            scatter(p, c0, g_a)
            gwait(p, c0 + 1, g_b, sem_b)
            gather(p, c0 + 2, g_a, sem_a)
            scatter(p, c0 + 1, g_b)
        gwait(p, W - 2, g_a, sem_a)
        gather(p, W - 1, g_b, sem_b)
        scatter(p, W - 2, g_a)
        gwait(p, W - 1, g_b, sem_b)
        pltpu.make_async_copy(src_hbm.at[pl.ds(nbase, W)], stg_src.at[1 - p],
                              sem_i).wait()
        pltpu.make_async_copy(dst_hbm.at[pl.ds(nbase, W)], stg_dst.at[1 - p],
                              sem_i).wait()
        gather(1 - p, 0, g_a, sem_a)
        scatter(p, W - 1, g_b)
        return 0

    lax.fori_loop(0, NWIN, window, 0)
    gwait(0, 0, g_a, sem_a)
    if with_cnt:
        def drain(c, _):
            pltpu.make_async_copy(ones_v, cnt_sh.at[stg_dst.at[0, 0]],
                                  sem_c).wait()
            return 0

        lax.fori_loop(0, NCH, drain, 0)
    plsc.subcore_barrier()

    r = sid * RPW
    pltpu.sync_copy(s_sh.at[pl.ds(r, RPW)], s_out.at[pl.ds(cid * N_PAD + r, RPW)])
    if with_cnt:
        pltpu.sync_copy(cnt_sh.at[pl.ds(sid * CPW, CPW)],
                        cnt_out.at[pl.ds(cid * CNT_PAD + sid * CPW, CPW)])


def _make_agg(with_cnt):
    out_type = [jax.ShapeDtypeStruct((NC * N_PAD, D), jnp.float32)]
    scratch = [
        pltpu.VMEM_SHARED((N_PAD, D), jnp.float32),
        pltpu.VMEM((2, W, CHUNK), jnp.int32),
        pltpu.VMEM((2, W, CHUNK), jnp.int32),
        pltpu.VMEM((CHUNK, D), jnp.float32),
        pltpu.VMEM((CHUNK, D), jnp.float32),
        pltpu.SemaphoreType.DMA,
        pltpu.SemaphoreType.DMA,
        pltpu.SemaphoreType.DMA,
        pltpu.SemaphoreType.DMA,
        pltpu.SemaphoreType.DMA,
    ]
    if with_cnt:
        out_type.append(jax.ShapeDtypeStruct((NC * CNT_PAD,), jnp.float32))
        scratch = ([scratch[0], pltpu.VMEM_SHARED((CNT_PAD,), jnp.float32)]
                   + scratch[1:-5]
                   + [pltpu.VMEM((CHUNK,), jnp.float32)]
                   + [pltpu.SemaphoreType.DMA] * 6)
    return pl.kernel(
        functools.partial(_agg_body, with_cnt),
        out_type=out_type,
        mesh=_mesh,
        scratch_types=scratch,
    )


def _sload(ref, idx):
    return ref[pl.ds(idx, 16)][0]


NCOARSE = (N + 15) // 16


def _pool_body(h_hbm, batch_hbm, coarse_hbm, ninf_hbm, out_hbm,
               coarse_v, bwin, rbuf, acc, sem):
    cid = lax.axis_index("c")
    sid = lax.axis_index("s")
    wid = cid * NS + sid
    g0 = wid * GPW
    pltpu.sync_copy(coarse_hbm, coarse_v.at[pl.ds(0, NCOARSE)])
    pltpu.sync_copy(ninf_hbm, acc)

    def lower_bound(tgt):
        pos = jnp.int32(0)
        w = 1024
        while w:
            cand = pos + w
            v = _sload(coarse_v, jnp.minimum(cand, NCOARSE) - 1)
            pos = jnp.where((cand <= NCOARSE) & (v < tgt), cand, pos)
            w //= 2
        return pos

    lo = jnp.maximum(lower_bound(g0) - 1, 0) * 16
    hi = jnp.minimum(lower_bound(g0 + GPW) * 16, N)

    nwin = (hi - lo + RB - 1) // RB

    def step(k, _):
        rc = pl.multiple_of(jnp.minimum(lo + k * RB, N - RB), 8)
        pltpu.sync_copy(h_hbm.at[pl.ds(rc, RB)], rbuf)
        pltpu.sync_copy(batch_hbm.at[pl.ds(rc, RB)], bwin.at[pl.ds(0, RB)])

        def row(i, _):
            g = _sload(bwin, i)
            gl = g - g0
            valid = (gl >= 0) & (gl < GPW)
            glc = jnp.clip(gl, 0, GPW - 1)
            for j in range(D // 16):
                cur = acc[glc, pl.ds(j * 16, 16)]
                new = jnp.maximum(cur, rbuf[i, pl.ds(j * 16, 16)])
                acc[glc, pl.ds(j * 16, 16)] = jnp.where(valid, new, cur)
            return 0

        lax.fori_loop(0, RB, row, 0)
        return 0

    lax.fori_loop(0, nwin, step, 0)
    pltpu.sync_copy(acc, out_hbm.at[pl.ds(g0, GPW)])


_pool = pl.kernel(
    _pool_body,
    out_type=jax.ShapeDtypeStruct((G, D), jnp.float32),
    mesh=_mesh,
    scratch_types=[
        pltpu.VMEM((NCOARSE + 31, ), jnp.int32),
        pltpu.VMEM((RB + 16,), jnp.int32),
        pltpu.VMEM((RB, D), jnp.float32),
        pltpu.VMEM((GPW, D), jnp.float32),
        pltpu.SemaphoreType.DMA,
    ],
)

_CONTRACT_T = (((1,), (1,)), ((), ()))


def _dense_block(apply_ln, s_ref, cnt_ref, x_ref, wl_ref, wr_ref, b_ref,
                 g_ref, bb_ref, o_ref):
    s = s_ref[0] + s_ref[1]
    cnt = cnt_ref[0] + cnt_ref[1]
    mean = s / jnp.maximum(cnt, 1.0)
    h = (lax.dot_general(mean, wl_ref[...], _CONTRACT_T,
                         preferred_element_type=jnp.float32)
         + lax.dot_general(x_ref[...], wr_ref[...], _CONTRACT_T,
                           preferred_element_type=jnp.float32)
         + b_ref[...])
    if apply_ln:
        mu = jnp.mean(h, axis=-1, keepdims=True)
        var = jnp.mean((h - mu) ** 2, axis=-1, keepdims=True)
        h = (h - mu) * lax.rsqrt(var + 1e-5) * g_ref[...] + bb_ref[...]
        h = jnp.maximum(h, 0.0)
    o_ref[...] = h


def _make_dense(apply_ln):
    return pl.pallas_call(
        functools.partial(_dense_block, apply_ln),
        grid=(N // RB1,),
        in_specs=[
            pl.BlockSpec((NC, RB1, D), lambda i: (0, i, 0)),
            pl.BlockSpec((NC, RB1, 1), lambda i: (0, i, 0)),
            pl.BlockSpec((RB1, D), lambda i: (i, 0)),
            pl.BlockSpec((D, D), lambda i: (0, 0)),
            pl.BlockSpec((D, D), lambda i: (0, 0)),
            pl.BlockSpec((1, D), lambda i: (0, 0)),
            pl.BlockSpec((1, D), lambda i: (0, 0)),
            pl.BlockSpec((1, D), lambda i: (0, 0)),
        ],
        out_specs=pl.BlockSpec((RB1, D), lambda i: (i, 0)),
        out_shape=jax.ShapeDtypeStruct((N, D), jnp.float32),
    )


def _head_block(p_ref, w_ref, b_ref, g_ref, bb_ref, o_ref):
    y = lax.dot_general(p_ref[...], w_ref[...], _CONTRACT_T,
                        preferred_element_type=jnp.float32) + b_ref[...]
    y = jnp.maximum(y, 0.0)
    mu = jnp.mean(y, axis=-1, keepdims=True)
    var = jnp.mean((y - mu) ** 2, axis=-1, keepdims=True)
    o_ref[...] = (y - mu) * lax.rsqrt(var + 1e-5) * g_ref[...] + bb_ref[...]


_head = pl.pallas_call(
    _head_block,
    out_shape=jax.ShapeDtypeStruct((G, D), jnp.float32),
)

_agg1 = _make_agg(True)
_agg2 = _make_agg(False)
_dense1 = _make_dense(True)
_dense2 = _make_dense(False)


def kernel(atom_features, edge_index, batch, W1l, b1, W1r, gn1, bn1,
           W2l, b2, W2r, Wout, bout, g_ln, b_ln):
    src2 = edge_index[0].reshape(E // CHUNK, CHUNK)
    dst2 = edge_index[1].reshape(E // CHUNK, CHUNK)
    zrow = jnp.zeros((N_PAD, D), jnp.float32)
    zcnt = jnp.zeros((CPW,), jnp.float32)
    ones = jnp.ones((CHUNK,), jnp.float32)
    ninf = jnp.full((GPW, D), float("-inf"), jnp.float32)

    s1, cnt = _agg1(atom_features, src2, dst2, zrow, zcnt, ones)
    s1 = s1.reshape(NC, N_PAD, D)
    cnt3 = cnt.reshape(NC, CNT_PAD, 1)
    b1r = b1.reshape(1, D)
    h1 = _dense1(s1, cnt3, atom_features, W1l, W1r, b1r,
                 gn1.reshape(1, D), bn1.reshape(1, D))
    (s2,) = _agg2(h1, src2, dst2, zrow)
    s2 = s2.reshape(NC, N_PAD, D)
    h2 = _dense2(s2, cnt3, h1, W2l, W2r, b2.reshape(1, D),
                 gn1.reshape(1, D), bn1.reshape(1, D))
    coarse = batch[::16]
    pooled = _pool(h2, batch, coarse, ninf)
    return _head(pooled, Wout, bout.reshape(1, D),
                 g_ln.reshape(1, D), b_ln.reshape(1, D))


# speedup vs baseline: 1.0749x; 1.0059x over previous
import functools

import jax
import jax.numpy as jnp
from jax import lax
from jax.experimental import pallas as pl
from jax.experimental.pallas import tpu as pltpu
from jax.experimental.pallas import tpu_sc as plsc

N = 10000
E = 320000
G = 256
D = 128
NC = 2
NS = 16
NW = NC * NS
EPW = E // NW
CHUNK = 125
NCH = EPW // CHUNK
W = 8
NWIN = NCH // W
N_PAD = 10112
RPW = N_PAD // NS
CNT_PAD = 10240
CPW = CNT_PAD // NS
GPW = G // NW
RB = 128
RB1 = 2000

_mesh = plsc.VectorSubcoreMesh(core_axis_name="c", subcore_axis_name="s")


def _agg_body(with_cnt, *refs):
    if with_cnt:
        (x_hbm, src_hbm, dst_hbm, zrow_hbm, zcnt_hbm, ones_hbm,
         s_out, cnt_out, s_sh, cnt_sh, stg_src, stg_dst, g_a, g_b, ones_v,
         sem_a, sem_b, sem_i, sem_sa, sem_sb, sem_c) = refs
    else:
        (x_hbm, src_hbm, dst_hbm, zrow_hbm,
         s_out, s_sh, stg_src, stg_dst, g_a, g_b,
         sem_a, sem_b, sem_i, sem_sa, sem_sb) = refs
    cid = lax.axis_index("c")
    sid = lax.axis_index("s")
    wid = cid * NS + sid

    pltpu.sync_copy(zrow_hbm.at[pl.ds(sid * RPW, RPW)],
                    s_sh.at[pl.ds(sid * RPW, RPW)])
    if with_cnt:
        pltpu.sync_copy(zcnt_hbm.at[pl.ds(sid * CPW, CPW)],
                        cnt_sh.at[pl.ds(sid * CPW, CPW)])
        pltpu.sync_copy(ones_hbm, ones_v)
    plsc.subcore_barrier()

    def gather(p, c, buf, sem):
        pltpu.async_copy(x_hbm.at[stg_src.at[p, c]], buf, sem)

    def gwait(p, c, buf, sem):
        pltpu.make_async_copy(x_hbm.at[stg_src.at[p, c]], buf, sem).wait()

    def scatter(p, c, buf):
        pltpu.sync_copy(buf, s_sh.at[stg_dst.at[p, c]], add=True)
        if with_cnt:
            pltpu.async_copy(ones_v, cnt_sh.at[stg_dst.at[p, c]], sem_c,
                             add=True)

    pltpu.sync_copy(src_hbm.at[pl.ds(wid * NCH, W)], stg_src.at[0])
    pltpu.sync_copy(dst_hbm.at[pl.ds(wid * NCH, W)], stg_dst.at[0])
    gather(0, 0, g_a, sem_a)

    def window(win, _):
        p = win % 2
        nxt = jnp.minimum(win + 1, NWIN - 1)
        nbase = wid * NCH + nxt * W
        pltpu.async_copy(src_hbm.at[pl.ds(nbase, W)], stg_src.at[1 - p], sem_i)
        pltpu.async_copy(dst_hbm.at[pl.ds(nbase, W)], stg_dst.at[1 - p], sem_i)
        for t in range(W // 2 - 1):
            c0 = 2 * t
            gwait(p, c0, g_a, sem_a)
            gather(p, c0 + 1, g_b, sem_b)
            scatter(p, c0, g_a)
            gwait(p, c0 + 1, g_b, sem_b)
            gather(p, c0 + 2, g_a, sem_a)
            scatter(p, c0 + 1, g_b)
        gwait(p, W - 2, g_a, sem_a)
        gather(p, W - 1, g_b, sem_b)
        scatter(p, W - 2, g_a)
        gwait(p, W - 1, g_b, sem_b)
        pltpu.make_async_copy(src_hbm.at[pl.ds(nbase, W)], stg_src.at[1 - p],
                              sem_i).wait()
        pltpu.make_async_copy(dst_hbm.at[pl.ds(nbase, W)], stg_dst.at[1 - p],
                              sem_i).wait()
        gather(1 - p, 0, g_a, sem_a)
        scatter(p, W - 1, g_b)
        return 0

    lax.fori_loop(0, NWIN, window, 0)
    gwait(0, 0, g_a, sem_a)
    if with_cnt:
        def drain(c, _):
            pltpu.make_async_copy(ones_v, cnt_sh.at[stg_dst.at[0, 0]],
                                  sem_c).wait()
            return 0

        lax.fori_loop(0, NCH, drain, 0)
    plsc.subcore_barrier()

    r = sid * RPW
    pltpu.sync_copy(s_sh.at[pl.ds(r, RPW)], s_out.at[pl.ds(cid * N_PAD + r, RPW)])
    if with_cnt:
        pltpu.sync_copy(cnt_sh.at[pl.ds(sid * CPW, CPW)],
                        cnt_out.at[pl.ds(cid * CNT_PAD + sid * CPW, CPW)])


def _make_agg(with_cnt):
    out_type = [jax.ShapeDtypeStruct((NC * N_PAD, D), jnp.float32)]
    scratch = [
        pltpu.VMEM_SHARED((N_PAD, D), jnp.float32),
        pltpu.VMEM((2, W, CHUNK), jnp.int32),
        pltpu.VMEM((2, W, CHUNK), jnp.int32),
        pltpu.VMEM((CHUNK, D), jnp.float32),
        pltpu.VMEM((CHUNK, D), jnp.float32),
        pltpu.SemaphoreType.DMA,
        pltpu.SemaphoreType.DMA,
        pltpu.SemaphoreType.DMA,
        pltpu.SemaphoreType.DMA,
        pltpu.SemaphoreType.DMA,
    ]
    if with_cnt:
        out_type.append(jax.ShapeDtypeStruct((NC * CNT_PAD,), jnp.float32))
        scratch = ([scratch[0], pltpu.VMEM_SHARED((CNT_PAD,), jnp.float32)]
                   + scratch[1:-5]
                   + [pltpu.VMEM((CHUNK,), jnp.float32)]
                   + [pltpu.SemaphoreType.DMA] * 6)
    return pl.kernel(
        functools.partial(_agg_body, with_cnt),
        out_type=out_type,
        mesh=_mesh,
        scratch_types=scratch,
    )


def _sload(ref, idx):
    return ref[pl.ds(idx, 16)][0]


NCOARSE = (N + 15) // 16


def _pool_body(h_hbm, batch_hbm, coarse_hbm, ninf_hbm, out_hbm,
               coarse_v, bwin, rbuf, acc, sem):
    cid = lax.axis_index("c")
    sid = lax.axis_index("s")
    wid = cid * NS + sid
    g0 = wid * GPW
    pltpu.sync_copy(coarse_hbm, coarse_v.at[pl.ds(0, NCOARSE)])
    pltpu.sync_copy(ninf_hbm, acc)

    def lower_bound(tgt):
        pos = jnp.int32(0)
        w = 1024
        while w:
            cand = pos + w
            v = _sload(coarse_v, jnp.minimum(cand, NCOARSE) - 1)
            pos = jnp.where((cand <= NCOARSE) & (v < tgt), cand, pos)
            w //= 2
        return pos

    lo = jnp.maximum(lower_bound(g0) - 1, 0) * 16
    hi = jnp.minimum(lower_bound(g0 + GPW) * 16, N)

    nwin = (hi - lo + RB - 1) // RB

    def step(k, _):
        rc = pl.multiple_of(jnp.minimum(lo + k * RB, N - RB), 8)
        pltpu.sync_copy(h_hbm.at[pl.ds(rc, RB)], rbuf)
        pltpu.sync_copy(batch_hbm.at[pl.ds(rc, RB)], bwin.at[pl.ds(0, RB)])

        def row(i, _):
            g = _sload(bwin, i)
            gl = g - g0
            valid = (gl >= 0) & (gl < GPW)
            glc = jnp.where(valid, gl, GPW)
            for j in range(D // 16):
                cur = acc[glc, pl.ds(j * 16, 16)]
                new = jnp.maximum(cur, rbuf[i, pl.ds(j * 16, 16)])
                acc[glc, pl.ds(j * 16, 16)] = new
            return 0

        lax.fori_loop(0, RB, row, 0)
        return 0

    lax.fori_loop(0, nwin, step, 0)
    pltpu.sync_copy(acc.at[pl.ds(0, GPW)], out_hbm.at[pl.ds(g0, GPW)])


_pool = pl.kernel(
    _pool_body,
    out_type=jax.ShapeDtypeStruct((G, D), jnp.float32),
    mesh=_mesh,
    scratch_types=[
        pltpu.VMEM((NCOARSE + 31, ), jnp.int32),
        pltpu.VMEM((RB + 16,), jnp.int32),
        pltpu.VMEM((RB, D), jnp.float32),
        pltpu.VMEM((GPW + 1, D), jnp.float32),
        pltpu.SemaphoreType.DMA,
    ],
)

_CONTRACT_T = (((1,), (1,)), ((), ()))


def _dense_block(apply_ln, s_ref, cnt_ref, x_ref, wl_ref, wr_ref, b_ref,
                 g_ref, bb_ref, o_ref):
    s = s_ref[0] + s_ref[1]
    cnt = cnt_ref[0] + cnt_ref[1]
    mean = s / jnp.maximum(cnt, 1.0)
    h = (lax.dot_general(mean, wl_ref[...], _CONTRACT_T,
                         preferred_element_type=jnp.float32)
         + lax.dot_general(x_ref[...], wr_ref[...], _CONTRACT_T,
                           preferred_element_type=jnp.float32)
         + b_ref[...])
    if apply_ln:
        mu = jnp.mean(h, axis=-1, keepdims=True)
        var = jnp.mean((h - mu) ** 2, axis=-1, keepdims=True)
        h = (h - mu) * lax.rsqrt(var + 1e-5) * g_ref[...] + bb_ref[...]
        h = jnp.maximum(h, 0.0)
    o_ref[...] = h


def _make_dense(apply_ln):
    return pl.pallas_call(
        functools.partial(_dense_block, apply_ln),
        grid=(N // RB1,),
        in_specs=[
            pl.BlockSpec((NC, RB1, D), lambda i: (0, i, 0)),
            pl.BlockSpec((NC, RB1, 1), lambda i: (0, i, 0)),
            pl.BlockSpec((RB1, D), lambda i: (i, 0)),
            pl.BlockSpec((D, D), lambda i: (0, 0)),
            pl.BlockSpec((D, D), lambda i: (0, 0)),
            pl.BlockSpec((1, D), lambda i: (0, 0)),
            pl.BlockSpec((1, D), lambda i: (0, 0)),
            pl.BlockSpec((1, D), lambda i: (0, 0)),
        ],
        out_specs=pl.BlockSpec((RB1, D), lambda i: (i, 0)),
        out_shape=jax.ShapeDtypeStruct((N, D), jnp.float32),
    )


def _head_block(p_ref, w_ref, b_ref, g_ref, bb_ref, o_ref):
    y = lax.dot_general(p_ref[...], w_ref[...], _CONTRACT_T,
                        preferred_element_type=jnp.float32) + b_ref[...]
    y = jnp.maximum(y, 0.0)
    mu = jnp.mean(y, axis=-1, keepdims=True)
    var = jnp.mean((y - mu) ** 2, axis=-1, keepdims=True)
    o_ref[...] = (y - mu) * lax.rsqrt(var + 1e-5) * g_ref[...] + bb_ref[...]


_head = pl.pallas_call(
    _head_block,
    out_shape=jax.ShapeDtypeStruct((G, D), jnp.float32),
)

_agg1 = _make_agg(True)
_agg2 = _make_agg(False)
_dense1 = _make_dense(True)
_dense2 = _make_dense(False)


def kernel(atom_features, edge_index, batch, W1l, b1, W1r, gn1, bn1,
           W2l, b2, W2r, Wout, bout, g_ln, b_ln):
    src2 = edge_index[0].reshape(E // CHUNK, CHUNK)
    dst2 = edge_index[1].reshape(E // CHUNK, CHUNK)
    zrow = jnp.zeros((N_PAD, D), jnp.float32)
    zcnt = jnp.zeros((CNT_PAD,), jnp.float32)
    ones = jnp.ones((CHUNK,), jnp.float32)
    ninf = jnp.full((GPW + 1, D), float("-inf"), jnp.float32)

    s1, cnt = _agg1(atom_features, src2, dst2, zrow, zcnt, ones)
    s1 = s1.reshape(NC, N_PAD, D)
    cnt3 = cnt.reshape(NC, CNT_PAD, 1)
    b1r = b1.reshape(1, D)
    h1 = _dense1(s1, cnt3, atom_features, W1l, W1r, b1r,
                 gn1.reshape(1, D), bn1.reshape(1, D))
    (s2,) = _agg2(h1, src2, dst2, zrow)
    s2 = s2.reshape(NC, N_PAD, D)
    h2 = _dense2(s2, cnt3, h1, W2l, W2r, b2.reshape(1, D),
                 gn1.reshape(1, D), bn1.reshape(1, D))
    coarse = batch[::16]
    pooled = _pool(h2, batch, coarse, ninf)
    return _head(pooled, Wout, bout.reshape(1, D),
                 g_ln.reshape(1, D), b_ln.reshape(1, D))
